# baseline (speedup 1.0000x reference)
import functools

import jax
import jax.numpy as jnp
from jax import lax
from jax.experimental import pallas as pl
from jax.experimental.pallas import tpu as pltpu
from jax.experimental.pallas import tpu_sc as plsc

BATCH = 16384
FIELDS = 26
EMB_DIM = 32
LANES = 16

NUM_CORES = 2
NUM_SUBCORES = 16
NUM_WORKERS = NUM_CORES * NUM_SUBCORES
NB = BATCH // NUM_WORKERS


@jax.jit
def _sc_gather(d, table):
    mesh = plsc.VectorSubcoreMesh(core_axis_name="c", subcore_axis_name="s")

    @functools.partial(
        pl.kernel,
        mesh=mesh,
        out_type=jax.ShapeDtypeStruct((BATCH, FIELDS, EMB_DIM), jnp.float32),
        scratch_types=[
            pltpu.VMEM((NB, FIELDS), jnp.int32),
            pltpu.VMEM((FIELDS * NB,), jnp.int32),
            pltpu.VMEM((NB, 1, EMB_DIM), jnp.float32),
            pltpu.SemaphoreType.DMA,
        ],
        compiler_params=pltpu.CompilerParams(
            use_tc_tiling_on_sc=False, needs_layout_passes=False
        ),
    )
    def k(table_hbm, idx_hbm, out_hbm, idx2d, idx1d, rows_v, sem):
        wid = lax.axis_index("s") * NUM_CORES + lax.axis_index("c")
        b0 = wid * NB

        pltpu.sync_copy(idx_hbm.at[pl.ds(b0, NB), :], idx2d)

        lane = lax.iota(jnp.int32, LANES)
        lo_tgt = lane * NB
        hi_tgt = (lane + 10) * NB

        def body(j, _):
            row = idx2d.at[j]
            plsc.store_scatter(idx1d, [lo_tgt + j], row[pl.ds(0, LANES)])
            plsc.store_scatter(idx1d, [hi_tgt + j], row[pl.ds(10, LANES)])
            return ()

        lax.fori_loop(0, NB, body, (), unroll=8)

        for f in range(FIELDS):
            pltpu.async_copy(
                table_hbm.at[idx1d.at[pl.ds(f * NB, NB)]],
                rows_v.at[:, 0, :],
                sem,
            ).wait()
            pltpu.sync_copy(
                rows_v, out_hbm.at[pl.ds(b0, NB), pl.ds(f, 1), :]
            )

    return k(table, d)


def kernel(d, embedding):
    return _sc_gather(d.astype(jnp.int32), embedding)


# speedup vs baseline: 1.2804x; 1.2804x over previous
import functools

import jax
import jax.numpy as jnp
from jax import lax
from jax.experimental import pallas as pl
from jax.experimental.pallas import tpu as pltpu
from jax.experimental.pallas import tpu_sc as plsc

BATCH = 16384
FIELDS = 26
EMB_DIM = 32
LANES = 16

NUM_CORES = 2
NUM_SUBCORES = 16
NUM_WORKERS = NUM_CORES * NUM_SUBCORES
NB = BATCH // NUM_WORKERS


@jax.jit
def _sc_gather(d, table):
    mesh = plsc.VectorSubcoreMesh(core_axis_name="c", subcore_axis_name="s")

    @functools.partial(
        pl.kernel,
        mesh=mesh,
        out_type=jax.ShapeDtypeStruct((BATCH, 32, 128), jnp.float32),
        scratch_types=[
            pltpu.VMEM((NB, FIELDS), jnp.int32),
            pltpu.VMEM((FIELDS * NB,), jnp.int32),
            pltpu.VMEM((2, NB, 1, EMB_DIM), jnp.float32),
            pltpu.SemaphoreType.DMA,
            pltpu.SemaphoreType.DMA,
        ],
        compiler_params=pltpu.CompilerParams(
            use_tc_tiling_on_sc=False, needs_layout_passes=False
        ),
    )
    def k(table_hbm, idx_hbm, out_hbm, idx2d, idx1d, rows_v, sem0, sem1):
        wid = lax.axis_index("s") * NUM_CORES + lax.axis_index("c")
        b0 = wid * NB

        pltpu.sync_copy(idx_hbm.at[pl.ds(b0, NB), :], idx2d)

        lane = lax.iota(jnp.int32, LANES)
        lo_tgt = lane * NB
        hi_tgt = (lane + 10) * NB

        def body(j, _):
            row = idx2d.at[j]
            plsc.store_scatter(idx1d, [lo_tgt + j], row[pl.ds(0, LANES)])
            plsc.store_scatter(idx1d, [hi_tgt + j], row[pl.ds(10, LANES)])
            return ()

        lax.fori_loop(0, NB, body, (), unroll=8)

        sems = (sem0, sem1)
        descs = [None, None]
        for f in range(FIELDS + 1):
            if f < FIELDS:
                p = f % 2
                descs[p] = pltpu.async_copy(
                    table_hbm.at[idx1d.at[pl.ds(f * NB, NB)]],
                    rows_v.at[p, :, 0, :],
                    sems[p],
                )
            if f >= 1:
                q = (f - 1) % 2
                descs[q].wait()
                pltpu.sync_copy(
                    rows_v.at[q],
                    out_hbm.at[
                        pl.ds(b0, NB), pl.ds(f - 1, 1), pl.ds(0, EMB_DIM)
                    ],
                )

    return k(table, d)


def kernel(d, embedding):
    out = _sc_gather(d.astype(jnp.int32), embedding)
    return out[:, :FIELDS, :EMB_DIM]
